# baseline (speedup 1.0000x reference)
import functools

import jax
import jax.numpy as jnp
import numpy as np
from jax import lax
from jax.experimental import pallas as pl
from jax.experimental.pallas import tpu as pltpu
from jax.experimental.pallas import tpu_sc as plsc

B = 16384
D_EMB = 64
N_CONT = 64
HIDDEN = 32
N_STACKS = 100000
PACK = 128 // HIDDEN
S = 25088
CHUNK = 128
BLK1 = S // 4

_S_ALL = np.arange(B, dtype=np.int32)
_SIDX = (PACK * (_S_ALL % (B // PACK)) + _S_ALL // (B // PACK)).reshape(
    B // CHUNK, CHUNK
)


def _premul_body(
    t0_ref, t1_ref, t2_ref, t3_ref, wet_ref, ct_ref, wct_ref, bt_ref,
    o_hbm, hct_ref, buf, sem,
):
    nlb = S // BLK1
    i = pl.program_id(0)
    wet = wet_ref[...]
    zs = [
        lax.dot_general(
            wet,
            t_ref[...],
            (((1,), (0,)), ((), ())),
            preferred_element_type=jnp.float32,
        )
        for t_ref in (t0_ref, t1_ref, t2_ref, t3_ref)
    ]
    z = jnp.concatenate(zs, axis=0)

    hct_ref[...] = lax.dot_general(
        wct_ref[...],
        ct_ref[...],
        (((1,), (0,)), ((), ())),
        preferred_element_type=jnp.float32,
    ) + bt_ref[...]

    slot = lax.rem(i, 2)
    prev_slot = lax.rem(i + 1, 2)

    @pl.when(i > 0)
    def _():
        pltpu.make_async_copy(
            buf.at[prev_slot],
            o_hbm.at[pl.ds((i - 1) * BLK1, BLK1), :],
            sem,
        ).wait()

    buf[slot] = z.T
    cp = pltpu.make_async_copy(
        buf.at[slot],
        o_hbm.at[pl.ds(i * BLK1, BLK1), :],
        sem,
    )
    cp.start()

    @pl.when(i == nlb - 1)
    def _():
        cp.wait()


def _premul(tt, wet, ct, wct, bt):
    nlb = S // BLK1
    grid = (nlb,)

    def tt_spec(k):
        return pl.BlockSpec((D_EMB, BLK1), lambda i, _k=k: (0, _k * nlb + i))

    return pl.pallas_call(
        _premul_body,
        grid=grid,
        in_specs=[
            tt_spec(0),
            tt_spec(1),
            tt_spec(2),
            tt_spec(3),
            pl.BlockSpec((HIDDEN, D_EMB), lambda i: (0, 0)),
            pl.BlockSpec((N_CONT, B // nlb), lambda i: (0, i)),
            pl.BlockSpec((HIDDEN, N_CONT), lambda i: (0, 0)),
            pl.BlockSpec((HIDDEN, 1), lambda i: (0, 0)),
        ],
        out_specs=[
            pl.BlockSpec(memory_space=pltpu.MemorySpace.HBM),
            pl.BlockSpec((HIDDEN, B // nlb), lambda i: (0, i)),
        ],
        out_shape=[
            jax.ShapeDtypeStruct((S, PACK * HIDDEN), jnp.float32),
            jax.ShapeDtypeStruct((HIDDEN, B), jnp.float32),
        ],
        scratch_shapes=[
            pltpu.VMEM((2, BLK1, PACK * HIDDEN), jnp.float32),
            pltpu.SemaphoreType.DMA,
        ],
    )(tt, tt, tt, tt, wet, ct, wct, bt)


def _make_gather():
    info = plsc.get_sparse_core_info()
    nw = info.num_cores * info.num_subcores
    b_per_w = B // nw
    n_ch = b_per_w // CHUNK

    mesh = plsc.VectorSubcoreMesh(core_axis_name="c", subcore_axis_name="s")

    @functools.partial(
        pl.kernel,
        mesh=mesh,
        out_type=jax.ShapeDtypeStruct((B, HIDDEN), jnp.float32),
        scratch_types=[
            pltpu.VMEM((n_ch, CHUNK), jnp.int32),
            pltpu.VMEM((n_ch, CHUNK), jnp.int32),
            pltpu.VMEM((b_per_w, HIDDEN), jnp.float32),
            pltpu.SemaphoreType.DMA,
            pltpu.SemaphoreType.DMA,
        ],
        compiler_params=pltpu.CompilerParams(use_tc_tiling_on_sc=False),
    )
    def gather_kernel(y_hbm, idx_hbm, sidx_hbm, out_hbm, idx_v, sidx_v, rows_v, sem, sem2):
        wid = lax.axis_index("s") * info.num_cores + lax.axis_index("c")
        pltpu.sync_copy(idx_hbm.at[pl.ds(wid * n_ch, n_ch)], idx_v)
        pltpu.sync_copy(sidx_hbm.at[pl.ds(wid * n_ch, n_ch)], sidx_v)
        copies = []
        for j in range(n_ch):
            copies.append(
                pltpu.async_copy(
                    y_hbm.at[idx_v.at[j]],
                    rows_v.at[pl.ds(j * CHUNK, CHUNK)],
                    sem,
                )
            )
        scatters = []
        for j in range(n_ch):
            copies[j].wait()
            scatters.append(
                pltpu.async_copy(
                    rows_v.at[pl.ds(j * CHUNK, CHUNK)],
                    out_hbm.at[sidx_v.at[j]],
                    sem2,
                )
            )
        for cp in scatters:
            cp.wait()

    return gather_kernel


def _finish_body(g_ref, hct_ref, o_ref):
    gt = jnp.concatenate(
        [g_ref[:, a * HIDDEN:(a + 1) * HIDDEN].T for a in range(PACK)],
        axis=1,
    )
    o_ref[...] = jnp.maximum(gt + hct_ref[...], 0.0)


def _finish(g4s, hct):
    return pl.pallas_call(
        _finish_body,
        grid=(1,),
        in_specs=[
            pl.BlockSpec((B // PACK, PACK * HIDDEN), lambda i: (0, 0)),
            pl.BlockSpec((HIDDEN, B), lambda i: (0, 0)),
        ],
        out_specs=pl.BlockSpec((HIDDEN, B), lambda i: (0, 0)),
        out_shape=jax.ShapeDtypeStruct((HIDDEN, B), jnp.float32),
    )(g4s, hct)


def kernel(stack_code, cont_feats, emb_table, W, b):
    idx = stack_code.astype(jnp.int32)
    idx2 = (PACK * lax.rem(idx, S) + lax.div(idx, S)).reshape(B // CHUNK, CHUNK)
    sidx = jnp.asarray(_SIDX)

    tt = emb_table.T
    wet = W.T[:, :D_EMB]
    ct = cont_feats.T
    wct = W.T[:, D_EMB:]
    bt = b.reshape(HIDDEN, 1)

    y_packed, hct = _premul(tt, wet, ct, wct, bt)
    yv = y_packed.reshape(PACK * S, HIDDEN)

    g = _make_gather()(yv, idx2, sidx)
    g4s = g.reshape(B // PACK, PACK * HIDDEN)

    out_t = _finish(g4s, hct)
    return out_t.T


# speedup vs baseline: 1.1062x; 1.1062x over previous
import functools

import jax
import jax.numpy as jnp
import numpy as np
from jax import lax
from jax.experimental import pallas as pl
from jax.experimental.pallas import tpu as pltpu
from jax.experimental.pallas import tpu_sc as plsc

B = 16384
D_EMB = 64
N_CONT = 64
HIDDEN = 32
N_STACKS = 100000
PACK = 128 // HIDDEN
S = 25088
CHUNK = 128
BLK1 = S // 4

_S_ALL = np.arange(B, dtype=np.int32)
_SIDX = (PACK * (_S_ALL % (B // PACK)) + _S_ALL // (B // PACK)).reshape(
    B // CHUNK, CHUNK
)


def _premul_body(t0_ref, t1_ref, t2_ref, t3_ref, wet_ref, o_hbm, buf, sem):
    nlb = S // BLK1
    i = pl.program_id(0)
    wet = wet_ref[...]
    zs = [
        lax.dot_general(
            wet,
            t_ref[...],
            (((1,), (0,)), ((), ())),
            preferred_element_type=jnp.float32,
        )
        for t_ref in (t0_ref, t1_ref, t2_ref, t3_ref)
    ]
    z = jnp.concatenate(zs, axis=0)

    slot = lax.rem(i, 2)
    prev_slot = lax.rem(i + 1, 2)

    @pl.when(i > 0)
    def _():
        pltpu.make_async_copy(
            buf.at[prev_slot],
            o_hbm.at[pl.ds((i - 1) * BLK1, BLK1), :],
            sem,
        ).wait()

    buf[slot] = z.T
    cp = pltpu.make_async_copy(
        buf.at[slot],
        o_hbm.at[pl.ds(i * BLK1, BLK1), :],
        sem,
    )
    cp.start()

    @pl.when(i == nlb - 1)
    def _():
        cp.wait()


def _premul(tt, wet):
    nlb = S // BLK1
    grid = (nlb,)

    def tt_spec(k):
        return pl.BlockSpec((D_EMB, BLK1), lambda i, _k=k: (0, _k * nlb + i))

    return pl.pallas_call(
        _premul_body,
        grid=grid,
        in_specs=[
            tt_spec(0),
            tt_spec(1),
            tt_spec(2),
            tt_spec(3),
            pl.BlockSpec((HIDDEN, D_EMB), lambda i: (0, 0)),
        ],
        out_specs=pl.BlockSpec(memory_space=pltpu.MemorySpace.HBM),
        out_shape=jax.ShapeDtypeStruct((S, PACK * HIDDEN), jnp.float32),
        scratch_shapes=[
            pltpu.VMEM((2, BLK1, PACK * HIDDEN), jnp.float32),
            pltpu.SemaphoreType.DMA,
        ],
    )(tt, tt, tt, tt, wet)


def _make_gather():
    info = plsc.get_sparse_core_info()
    nw = info.num_cores * info.num_subcores
    b_per_w = B // nw
    n_ch = b_per_w // CHUNK

    mesh = plsc.VectorSubcoreMesh(core_axis_name="c", subcore_axis_name="s")

    @functools.partial(
        pl.kernel,
        mesh=mesh,
        out_type=jax.ShapeDtypeStruct((B, HIDDEN), jnp.float32),
        scratch_types=[
            pltpu.VMEM((n_ch, CHUNK), jnp.int32),
            pltpu.VMEM((n_ch, CHUNK), jnp.int32),
            pltpu.VMEM((b_per_w, HIDDEN), jnp.float32),
            pltpu.SemaphoreType.DMA,
            pltpu.SemaphoreType.DMA,
        ],
        compiler_params=pltpu.CompilerParams(use_tc_tiling_on_sc=False),
    )
    def gather_kernel(y_hbm, idx_hbm, sidx_hbm, out_hbm, idx_v, sidx_v, rows_v, sem, sem2):
        wid = lax.axis_index("s") * info.num_cores + lax.axis_index("c")
        pltpu.sync_copy(idx_hbm.at[pl.ds(wid * n_ch, n_ch)], idx_v)
        pltpu.sync_copy(sidx_hbm.at[pl.ds(wid * n_ch, n_ch)], sidx_v)
        copies = []
        for j in range(n_ch):
            copies.append(
                pltpu.async_copy(
                    y_hbm.at[idx_v.at[j]],
                    rows_v.at[pl.ds(j * CHUNK, CHUNK)],
                    sem,
                )
            )
        scatters = []
        for j in range(n_ch):
            copies[j].wait()
            scatters.append(
                pltpu.async_copy(
                    rows_v.at[pl.ds(j * CHUNK, CHUNK)],
                    out_hbm.at[sidx_v.at[j]],
                    sem2,
                )
            )
        for cp in scatters:
            cp.wait()

    return gather_kernel


def _finish_body(g_ref, ct_ref, wct_ref, bt_ref, o_ref):
    hct = lax.dot_general(
        wct_ref[...],
        ct_ref[...],
        (((1,), (0,)), ((), ())),
        preferred_element_type=jnp.float32,
    )
    J = B // PACK
    bias = bt_ref[...]
    eye = jnp.eye(HIDDEN, dtype=jnp.float32)
    for a in range(PACK):
        ga = g_ref[:, a * HIDDEN:(a + 1) * HIDDEN]
        gta = lax.dot_general(
            eye, ga, (((1,), (1,)), ((), ())),
            preferred_element_type=jnp.float32,
        )
        o_ref[:, a * J:(a + 1) * J] = jnp.maximum(
            gta + hct[:, a * J:(a + 1) * J] + bias, 0.0
        )


def _finish(g4s, ct, wct, bt):
    return pl.pallas_call(
        _finish_body,
        grid=(1,),
        in_specs=[
            pl.BlockSpec((B // PACK, PACK * HIDDEN), lambda i: (0, 0)),
            pl.BlockSpec((N_CONT, B), lambda i: (0, 0)),
            pl.BlockSpec((HIDDEN, N_CONT), lambda i: (0, 0)),
            pl.BlockSpec((HIDDEN, 1), lambda i: (0, 0)),
        ],
        out_specs=pl.BlockSpec((HIDDEN, B), lambda i: (0, 0)),
        out_shape=jax.ShapeDtypeStruct((HIDDEN, B), jnp.float32),
    )(g4s, ct, wct, bt)


def kernel(stack_code, cont_feats, emb_table, W, b):
    idx = stack_code.astype(jnp.int32)
    idx2 = (PACK * lax.rem(idx, S) + lax.div(idx, S)).reshape(B // CHUNK, CHUNK)
    sidx = jnp.asarray(_SIDX)

    tt = emb_table.T
    wet = W.T[:, :D_EMB]
    y_packed = _premul(tt, wet)
    yv = y_packed.reshape(PACK * S, HIDDEN)

    g = _make_gather()(yv, idx2, sidx)
    g4s = g.reshape(B // PACK, PACK * HIDDEN)

    ct = cont_feats.T
    wct = W.T[:, D_EMB:]
    bt = b.reshape(HIDDEN, 1)

    out_t = _finish(g4s, ct, wct, bt)
    return out_t.T
